# Initial kernel scaffold; baseline (speedup 1.0000x reference)
import jax
import jax.numpy as jnp
from jax.experimental import pallas as pl


def kernel(x):
    raise NotImplementedError("write your pallas kernel here")


# speedup vs baseline: 10.3093x; 10.3093x over previous
import functools
import math

import jax
import jax.numpy as jnp
from jax import lax
from jax.experimental import pallas as pl
from jax.experimental.pallas import tpu as pltpu
from jax.experimental.pallas import tpu_sc as plsc

N_SAMPLES = 1048576
N_BINS = 1024
X_MIN, X_MAX = -4.0, 4.0
SIGMA = (X_MAX - X_MIN) / N_BINS
DELTA = (X_MAX - X_MIN) / (N_BINS - 1)
RHO = DELTA / SIGMA
RHO2 = RHO * RHO
W = 4

NC, NS, L = 2, 16, 16
NW = NC * NS
CHUNK = N_SAMPLES // NW
NVEC = CHUNK // L
NBLK = N_BINS // L

SCALE = 1.0 / (N_SAMPLES * SIGMA * math.sqrt(2.0 * math.pi))
C_RATIO = [math.exp(-RHO2 * (k + 0.5)) for k in range(-W, W)]


def _sc_body(x_hbm, part_hbm, x_v, acc_v, part_v):
    wid = lax.axis_index("s") * NC + lax.axis_index("c")
    base = wid * CHUNK
    pltpu.sync_copy(x_hbm.at[pl.ds(base, CHUNK)], x_v)

    zero = jnp.zeros((L,), jnp.float32)
    rows = lax.iota(jnp.int32, L) * N_BINS

    def zero_blk(b, carry):
        for r in range(L):
            acc_v[pl.ds(pl.multiple_of(r * N_BINS + b * L, L), L)] = zero
        return carry

    lax.fori_loop(0, NBLK, zero_blk, 0)

    def sample_vec(i, carry):
        xv = x_v[pl.ds(pl.multiple_of(i * L, L), L)]
        t = (xv - X_MIN) * (1.0 / DELTA)
        j0 = (t + 0.5).astype(jnp.int32)
        u = t - j0.astype(jnp.float32)
        g = jnp.exp(RHO2 * u)
        w0 = u + W
        v = jnp.exp((-0.5 * RHO2) * (w0 * w0))
        for k in range(-W, W + 1):
            idx = j0 + k
            m = (idx >= 0) & (idx < N_BINS)
            idxc = jnp.minimum(jnp.maximum(idx, 0), N_BINS - 1)
            plsc.addupdate_scatter(acc_v, [rows + idxc], v, mask=m)
            if k < W:
                v = v * (g * C_RATIO[k + W])
        return carry

    lax.fori_loop(0, NVEC, sample_vec, 0)

    def reduce_blk(b, carry):
        tot = acc_v[pl.ds(pl.multiple_of(b * L, L), L)]
        for r in range(1, L):
            tot = tot + acc_v[pl.ds(pl.multiple_of(r * N_BINS + b * L, L), L)]
        part_v[pl.ds(pl.multiple_of(b * L, L), L)] = tot
        return carry

    lax.fori_loop(0, NBLK, reduce_blk, 0)
    pltpu.sync_copy(part_v, part_hbm.at[wid])


_sc_hist = functools.partial(
    pl.kernel,
    out_type=jax.ShapeDtypeStruct((NW, N_BINS), jnp.float32),
    mesh=plsc.VectorSubcoreMesh(core_axis_name="c", subcore_axis_name="s"),
    scratch_types=[
        pltpu.VMEM((CHUNK,), jnp.float32),
        pltpu.VMEM((L * N_BINS,), jnp.float32),
        pltpu.VMEM((N_BINS,), jnp.float32),
    ],
    compiler_params=pltpu.CompilerParams(needs_layout_passes=False),
)(_sc_body)


def _tc_reduce(p_ref, o_ref):
    o_ref[...] = jnp.sum(p_ref[...], axis=0, keepdims=True) * SCALE


@jax.jit
def kernel(x):
    partials = _sc_hist(x)
    hist = pl.pallas_call(
        _tc_reduce,
        out_shape=jax.ShapeDtypeStruct((1, N_BINS), jnp.float32),
    )(partials)
    return hist.reshape(N_BINS)
